# baseline (speedup 1.0000x reference)
import jax
import jax.numpy as jnp
from jax import lax
from jax.experimental import pallas as pl
from jax.experimental.pallas import tpu as pltpu
from jax.experimental.pallas import tpu_sc as plsc

BATCH = 4096
SEQ = 100
DIM = 128
NC = 2
NS = 16
L = 16
NW = NC * NS
CHUNK = BATCH // NW
NBUF = 5


def _body(ids_hbm, table_hbm, pos_hbm, cls_hbm, out_hbm,
          pos_v, ids_all, obuf, cls_s, table_s, gsems, osems, csem,
          tsem, isem, psem):
    sid = lax.axis_index("s")
    wid = sid * NC + lax.axis_index("c")
    cbase = wid * CHUNK

    @pl.when(sid == 0)
    def _stage():
        pltpu.async_copy(table_hbm, table_s, tsem)

    pltpu.async_copy(ids_hbm.at[:, pl.ds(cbase, CHUNK)], ids_all, isem)
    pltpu.async_copy(pos_hbm, pos_v, psem)

    @pl.when(sid == 0)
    def _cls_build():
        pltpu.sync_copy(cls_hbm, obuf.at[0, pl.ds(0, 1)])
        cls_regs = [obuf[0, 0, pl.ds(j * L, L)] for j in range(DIM // L)]

        @pl.loop(1, CHUNK)
        def cls_fill(r):
            for j in range(DIM // L):
                obuf[0, r, pl.ds(j * L, L)] = cls_regs[j]

        pltpu.sync_copy(obuf.at[0], cls_s)

    pltpu.make_async_copy(ids_hbm.at[:, pl.ds(cbase, CHUNK)], ids_all, isem).wait()
    pltpu.make_async_copy(pos_hbm, pos_v, psem).wait()

    @pl.when(sid == 0)
    def _stage_wait():
        pltpu.make_async_copy(table_hbm, table_s, tsem).wait()

    plsc.subcore_barrier()
    pltpu.async_copy(cls_s, out_hbm.at[0, pl.ds(cbase, CHUNK)], csem)

    def fire_g(c, b):
        pltpu.async_copy(table_s.at[ids_all.at[c - 1]], obuf.at[b], gsems[b])

    def wait_g(c, b):
        pltpu.make_async_copy(
            table_s.at[ids_all.at[c - 1]], obuf.at[b], gsems[b]).wait()

    def fire_out(c, b):
        pltpu.async_copy(
            obuf.at[b], out_hbm.at[c, pl.ds(cbase, CHUNK)], osems[b])

    def wait_out(c, b):
        pltpu.make_async_copy(
            obuf.at[b], out_hbm.at[c, pl.ds(cbase, CHUNK)], osems[b]).wait()

    def compute(c, b):
        p = [pos_v[c - 1, pl.ds(j * L, L)] for j in range(DIM // L)]

        @plsc.parallel_loop(0, CHUNK, unroll=8)
        def add_pos(r):
            for j in range(DIM // L):
                v = obuf[b, r, pl.ds(j * L, L)]
                obuf[b, r, pl.ds(j * L, L)] = v + p[j]

    def step(c, b, do_wait_out, do_fire_g):
        if do_wait_out:
            wait_out(c + 2 - NBUF, (b + 2) % NBUF)
        if do_fire_g:
            fire_g(c + 2, (b + 2) % NBUF)
        wait_g(c, b)
        compute(c, b)
        fire_out(c, b)

    fire_g(1, 0)
    fire_g(2, 1)
    step(1, 0, do_wait_out=False, do_fire_g=True)
    step(2, 1, do_wait_out=False, do_fire_g=True)
    step(3, 2, do_wait_out=False, do_fire_g=True)
    step(4, 3, do_wait_out=True, do_fire_g=True)
    step(5, 4, do_wait_out=True, do_fire_g=True)

    @pl.loop(6, SEQ - 4, step=NBUF)
    def main(i):
        for k in range(NBUF):
            step(i + k, k, do_wait_out=True, do_fire_g=True)

    step(SEQ - 4, 0, do_wait_out=True, do_fire_g=True)
    step(SEQ - 3, 1, do_wait_out=True, do_fire_g=True)
    step(SEQ - 2, 2, do_wait_out=True, do_fire_g=True)
    step(SEQ - 1, 3, do_wait_out=True, do_fire_g=False)
    step(SEQ, 4, do_wait_out=True, do_fire_g=False)
    wait_out(SEQ - 2, 2)
    wait_out(SEQ - 1, 3)
    wait_out(SEQ, 4)
    pltpu.make_async_copy(cls_s, out_hbm.at[0, pl.ds(cbase, CHUNK)], csem).wait()


def kernel(bin_ids, bin_table, pos_table, cls_table):
    ids_t = jnp.transpose(bin_ids)
    mesh = plsc.VectorSubcoreMesh(
        core_axis_name="c", subcore_axis_name="s",
        num_cores=NC, num_subcores=NS,
    )
    f = pl.kernel(
        _body,
        out_type=jax.ShapeDtypeStruct((SEQ + 1, BATCH, DIM), jnp.float32),
        mesh=mesh,
        scratch_types=[
            pltpu.VMEM((SEQ, DIM), jnp.float32),
            pltpu.VMEM((SEQ, CHUNK), jnp.int32),
            pltpu.VMEM((NBUF, CHUNK, DIM), jnp.float32),
            pltpu.VMEM_SHARED((CHUNK, DIM), jnp.float32),
            pltpu.VMEM_SHARED((1000, DIM), jnp.float32),

            [pltpu.SemaphoreType.DMA] * NBUF,
            [pltpu.SemaphoreType.DMA] * NBUF,
            pltpu.SemaphoreType.DMA,
            pltpu.SemaphoreType.DMA,
            pltpu.SemaphoreType.DMA,
            pltpu.SemaphoreType.DMA,
        ],
    )
    out_t = f(ids_t, bin_table, pos_table, cls_table)
    return jnp.transpose(out_t, (1, 0, 2))


# speedup vs baseline: 1.0104x; 1.0104x over previous
import jax
import jax.numpy as jnp
from jax import lax
from jax.experimental import pallas as pl
from jax.experimental.pallas import tpu as pltpu
from jax.experimental.pallas import tpu_sc as plsc

BATCH = 4096
SEQ = 100
DIM = 128
NC = 2
NS = 16
L = 16
NW = NC * NS
CHUNK = BATCH // NW
NBUF = 4


def _body(ids_hbm, table_hbm, pos_hbm, cls_hbm, out_hbm,
          pos_v, ids_all, obuf, cbuf, table_s, gsems, osems, csem,
          tsem, isem, psem):
    sid = lax.axis_index("s")
    wid = sid * NC + lax.axis_index("c")
    cbase = wid * CHUNK

    @pl.when(sid == 0)
    def _stage():
        pltpu.async_copy(table_hbm, table_s, tsem)

    pltpu.async_copy(ids_hbm.at[:, pl.ds(cbase, CHUNK)], ids_all, isem)
    pltpu.async_copy(pos_hbm, pos_v, psem)

    pltpu.sync_copy(cls_hbm, cbuf.at[pl.ds(0, 1)])
    cls_regs = [cbuf[0, pl.ds(j * L, L)] for j in range(DIM // L)]

    @pl.loop(1, CHUNK)
    def cls_fill(r):
        for j in range(DIM // L):
            cbuf[r, pl.ds(j * L, L)] = cls_regs[j]

    pltpu.async_copy(cbuf, out_hbm.at[0, pl.ds(cbase, CHUNK)], csem)

    pltpu.make_async_copy(ids_hbm.at[:, pl.ds(cbase, CHUNK)], ids_all, isem).wait()
    pltpu.make_async_copy(pos_hbm, pos_v, psem).wait()

    @pl.when(sid == 0)
    def _stage_wait():
        pltpu.make_async_copy(table_hbm, table_s, tsem).wait()

    plsc.subcore_barrier()

    def fire_g(c, b):
        pltpu.async_copy(table_s.at[ids_all.at[c - 1]], obuf.at[b], gsems[b])

    def wait_g(c, b):
        pltpu.make_async_copy(
            table_s.at[ids_all.at[c - 1]], obuf.at[b], gsems[b]).wait()

    def fire_out(c, b):
        pltpu.async_copy(
            obuf.at[b], out_hbm.at[c, pl.ds(cbase, CHUNK)], osems[b])

    def wait_out(c, b):
        pltpu.make_async_copy(
            obuf.at[b], out_hbm.at[c, pl.ds(cbase, CHUNK)], osems[b]).wait()

    def compute(c, b):
        p = [pos_v[c - 1, pl.ds(j * L, L)] for j in range(DIM // L)]

        @plsc.parallel_loop(0, CHUNK, unroll=8)
        def add_pos(r):
            for j in range(DIM // L):
                v = obuf[b, r, pl.ds(j * L, L)]
                obuf[b, r, pl.ds(j * L, L)] = v + p[j]

    def step(c, b, do_wait_out, do_fire_g):
        if do_wait_out:
            wait_out(c + 2 - NBUF, (b + 2) % NBUF)
        if do_fire_g:
            fire_g(c + 2, (b + 2) % NBUF)
        wait_g(c, b)
        compute(c, b)
        fire_out(c, b)

    fire_g(1, 0)
    fire_g(2, 1)
    step(1, 0, do_wait_out=False, do_fire_g=True)
    step(2, 1, do_wait_out=False, do_fire_g=True)

    @pl.loop(3, SEQ - 1, step=NBUF)
    def main(i):
        for k in range(NBUF):
            step(i + k, (k + 2) % NBUF, do_wait_out=True, do_fire_g=True)

    step(SEQ - 1, (SEQ - 2) % NBUF, do_wait_out=True, do_fire_g=False)
    step(SEQ, (SEQ - 1) % NBUF, do_wait_out=True, do_fire_g=False)
    wait_out(SEQ - 1, (SEQ - 2) % NBUF)
    wait_out(SEQ, (SEQ - 1) % NBUF)
    pltpu.make_async_copy(cbuf, out_hbm.at[0, pl.ds(cbase, CHUNK)], csem).wait()


def kernel(bin_ids, bin_table, pos_table, cls_table):
    ids_t = jnp.transpose(bin_ids)
    mesh = plsc.VectorSubcoreMesh(
        core_axis_name="c", subcore_axis_name="s",
        num_cores=NC, num_subcores=NS,
    )
    f = pl.kernel(
        _body,
        out_type=jax.ShapeDtypeStruct((SEQ + 1, BATCH, DIM), jnp.float32),
        mesh=mesh,
        scratch_types=[
            pltpu.VMEM((SEQ, DIM), jnp.float32),
            pltpu.VMEM((SEQ, CHUNK), jnp.int32),
            pltpu.VMEM((NBUF, CHUNK, DIM), jnp.float32),
            pltpu.VMEM((CHUNK, DIM), jnp.float32),
            pltpu.VMEM_SHARED((1000, DIM), jnp.float32),

            [pltpu.SemaphoreType.DMA] * NBUF,
            [pltpu.SemaphoreType.DMA] * NBUF,
            pltpu.SemaphoreType.DMA,
            pltpu.SemaphoreType.DMA,
            pltpu.SemaphoreType.DMA,
            pltpu.SemaphoreType.DMA,
        ],
    )
    out_t = f(ids_t, bin_table, pos_table, cls_table)
    return jnp.transpose(out_t, (1, 0, 2))
